# baseline (speedup 1.0000x reference)
import functools

import jax
import jax.numpy as jnp
from jax import lax
from jax.experimental import pallas as pl
from jax.experimental.pallas import tpu as pltpu
from jax.experimental.pallas import tpu_sc as plsc

VOCAB = 100000
EMB = 128
B = 4096
L = 20

NC = 2
NS = 16
NW = NC * NS
BPW = B // NW
G = 4
NG = BPW // G
NBUF = 8
LANE = 16


def _sc_gather_sum(idx_grouped, table):
    mesh = plsc.VectorSubcoreMesh(core_axis_name="c", subcore_axis_name="s")

    @functools.partial(
        pl.kernel,
        mesh=mesh,
        out_type=jax.ShapeDtypeStruct((B, EMB), jnp.float32),
        scratch_types=[
            pltpu.VMEM((NG, G * L), jnp.int32),
            pltpu.VMEM((NBUF, G * L, EMB), jnp.float32),
            pltpu.VMEM((BPW, EMB), jnp.float32),
            pltpu.SemaphoreType.DMA((NBUF,)),
        ],
    )
    def k(idx_hbm, table_hbm, out_hbm, idx_v, bufs, out_v, sems):
        wid = lax.axis_index("s") * NC + lax.axis_index("c")
        pltpu.sync_copy(idx_hbm.at[pl.ds(wid * NG, NG)], idx_v)

        for g in range(NBUF):
            pltpu.make_async_copy(
                table_hbm.at[idx_v.at[g]], bufs.at[g], sems.at[g]
            ).start()

        def body(g, _):
            sub = lax.rem(g, NBUF)
            pltpu.make_async_copy(
                table_hbm.at[idx_v.at[g]], bufs.at[sub], sems.at[sub]
            ).wait()
            for r in range(G):
                row = g * G + r
                for h in range(EMB // LANE):
                    acc = bufs[sub, r * L, pl.ds(h * LANE, LANE)]
                    for j in range(1, L):
                        acc = acc + bufs[sub, r * L + j, pl.ds(h * LANE, LANE)]
                    out_v[row, pl.ds(h * LANE, LANE)] = acc
            nxt = g + NBUF

            @pl.when(nxt < NG)
            def _():
                pltpu.make_async_copy(
                    table_hbm.at[idx_v.at[nxt]], bufs.at[sub], sems.at[sub]
                ).start()

            return 0

        lax.fori_loop(0, NG, body, 0)
        pltpu.sync_copy(out_v, out_hbm.at[pl.ds(wid * BPW, BPW)])

    return k(idx_grouped, table)


BB = 4096
BV = 512


def _mm_kernel(x_ref, w_ref, b_ref, o_ref):
    xb = x_ref[...].astype(jnp.bfloat16)
    wb = w_ref[...].astype(jnp.bfloat16)
    o_ref[...] = (
        lax.dot_general(
            xb,
            wb,
            (((1,), (1,)), ((), ())),
            preferred_element_type=jnp.float32,
        )
        + b_ref[...]
    )


def _tc_matmul(x, W, b2d):
    grid = (pl.cdiv(VOCAB, BV),)
    return pl.pallas_call(
        _mm_kernel,
        grid=grid,
        in_specs=[
            pl.BlockSpec((BB, EMB), lambda j: (0, 0)),
            pl.BlockSpec((BV, EMB), lambda j: (j, 0)),
            pl.BlockSpec((1, BV), lambda j: (0, j)),
        ],
        out_specs=pl.BlockSpec((BB, BV), lambda j: (0, j)),
        out_shape=jax.ShapeDtypeStruct((B, VOCAB), jnp.float32),
    )(x, W, b2d)


def kernel(inputs, emb_table, W, b):
    idx_grouped = inputs.astype(jnp.int32).reshape(NW * NG, G * L)
    summed = _sc_gather_sum(idx_grouped, emb_table)
    return _tc_matmul(summed, W, b.reshape(1, VOCAB))


# speedup vs baseline: 3.4854x; 3.4854x over previous
import functools

import jax
import jax.numpy as jnp
from jax import lax
from jax.experimental import pallas as pl
from jax.experimental.pallas import tpu as pltpu
from jax.experimental.pallas import tpu_sc as plsc

VOCAB = 100000
EMB = 128
B = 4096
L = 20

NC = 2
NS = 16
NW = NC * NS
BPW = B // NW
G = 4
NG = BPW // G
NBUF = 8
LANE = 16


def _sc_gather_sum(idx_grouped, table):
    mesh = plsc.VectorSubcoreMesh(core_axis_name="c", subcore_axis_name="s")

    @functools.partial(
        pl.kernel,
        mesh=mesh,
        out_type=jax.ShapeDtypeStruct((B, EMB), jnp.float32),
        scratch_types=[
            pltpu.VMEM((NG, G * L), jnp.int32),
            pltpu.VMEM((NBUF, G * L, EMB), jnp.float32),
            pltpu.VMEM((BPW, EMB), jnp.float32),
            pltpu.SemaphoreType.DMA((NBUF,)),
        ],
    )
    def k(idx_hbm, table_hbm, out_hbm, idx_v, bufs, out_v, sems):
        wid = lax.axis_index("s") * NC + lax.axis_index("c")
        pltpu.sync_copy(idx_hbm.at[pl.ds(wid * NG, NG)], idx_v)

        for g in range(NBUF):
            pltpu.make_async_copy(
                table_hbm.at[idx_v.at[g]], bufs.at[g], sems.at[g]
            ).start()

        def body(g, _):
            sub = lax.rem(g, NBUF)
            pltpu.make_async_copy(
                table_hbm.at[idx_v.at[g]], bufs.at[sub], sems.at[sub]
            ).wait()
            for r in range(G):
                row = g * G + r
                for h in range(EMB // LANE):
                    acc = bufs[sub, r * L, pl.ds(h * LANE, LANE)]
                    for j in range(1, L):
                        acc = acc + bufs[sub, r * L + j, pl.ds(h * LANE, LANE)]
                    out_v[row, pl.ds(h * LANE, LANE)] = acc
            nxt = g + NBUF

            @pl.when(nxt < NG)
            def _():
                pltpu.make_async_copy(
                    table_hbm.at[idx_v.at[nxt]], bufs.at[sub], sems.at[sub]
                ).start()

            return 0

        lax.fori_loop(0, NG, body, 0)
        pltpu.sync_copy(out_v, out_hbm.at[pl.ds(wid * BPW, BPW)])

    return k(idx_grouped, table)


BB = 4096
BV = 512


def _mm_kernel(x_ref, w_ref, b_ref, o_ref):
    xb = x_ref[...].astype(jnp.bfloat16)
    wb = w_ref[...].astype(jnp.bfloat16)
    o_ref[...] = (
        lax.dot_general(
            wb,
            xb,
            (((1,), (1,)), ((), ())),
            preferred_element_type=jnp.float32,
        )
        + b_ref[...].T
    )


def _tc_matmul(x, W, b2d):
    grid = (pl.cdiv(VOCAB, BV),)
    return pl.pallas_call(
        _mm_kernel,
        grid=grid,
        in_specs=[
            pl.BlockSpec((BB, EMB), lambda j: (0, 0)),
            pl.BlockSpec((BV, EMB), lambda j: (j, 0)),
            pl.BlockSpec((1, BV), lambda j: (0, j)),
        ],
        out_specs=pl.BlockSpec((BV, BB), lambda j: (j, 0)),
        out_shape=jax.ShapeDtypeStruct((VOCAB, B), jnp.float32),
    )(x, W, b2d)


def kernel(inputs, emb_table, W, b):
    idx_grouped = inputs.astype(jnp.int32).reshape(NW * NG, G * L)
    summed = _sc_gather_sum(idx_grouped, emb_table)
    out_t = _tc_matmul(summed, W, b.reshape(1, VOCAB))
    return out_t.T


# speedup vs baseline: 3.5641x; 1.0226x over previous
import functools

import jax
import jax.numpy as jnp
from jax import lax
from jax.experimental import pallas as pl
from jax.experimental.pallas import tpu as pltpu
from jax.experimental.pallas import tpu_sc as plsc

VOCAB = 100000
EMB = 128
B = 4096
L = 20

NC = 2
NS = 16
NW = NC * NS
BPW = B // NW
G = 4
NG = BPW // G
NBUF = 8
LANE = 16


def _sc_gather_sum(idx_grouped, table):
    mesh = plsc.VectorSubcoreMesh(core_axis_name="c", subcore_axis_name="s")

    @functools.partial(
        pl.kernel,
        mesh=mesh,
        out_type=jax.ShapeDtypeStruct((B, EMB), jnp.float32),
        scratch_types=[
            pltpu.VMEM((NG, G * L), jnp.int32),
            pltpu.VMEM((NBUF, G * L, EMB), jnp.float32),
            pltpu.VMEM((BPW, EMB), jnp.float32),
            pltpu.SemaphoreType.DMA((NBUF,)),
        ],
    )
    def k(idx_hbm, table_hbm, out_hbm, idx_v, bufs, out_v, sems):
        wid = lax.axis_index("s") * NC + lax.axis_index("c")
        pltpu.sync_copy(idx_hbm.at[pl.ds(wid * NG, NG)], idx_v)

        for g in range(NBUF):
            pltpu.make_async_copy(
                table_hbm.at[idx_v.at[g]], bufs.at[g], sems.at[g]
            ).start()

        def body(g, _):
            sub = lax.rem(g, NBUF)
            pltpu.make_async_copy(
                table_hbm.at[idx_v.at[g]], bufs.at[sub], sems.at[sub]
            ).wait()
            for r in range(G):
                row = g * G + r
                for h in range(EMB // LANE):
                    acc = bufs[sub, r * L, pl.ds(h * LANE, LANE)]
                    for j in range(1, L):
                        acc = acc + bufs[sub, r * L + j, pl.ds(h * LANE, LANE)]
                    out_v[row, pl.ds(h * LANE, LANE)] = acc
            nxt = g + NBUF

            @pl.when(nxt < NG)
            def _():
                pltpu.make_async_copy(
                    table_hbm.at[idx_v.at[nxt]], bufs.at[sub], sems.at[sub]
                ).start()

            return 0

        lax.fori_loop(0, NG, body, 0)
        pltpu.sync_copy(out_v, out_hbm.at[pl.ds(wid * BPW, BPW)])

    return k(idx_grouped, table)


BB = 4096
BV = 1024


def _mm_kernel(x_ref, w_ref, b_ref, o_ref):
    xb = x_ref[...].astype(jnp.bfloat16)
    wb = w_ref[...].astype(jnp.bfloat16)
    o_ref[...] = (
        lax.dot_general(
            wb,
            xb,
            (((1,), (1,)), ((), ())),
            preferred_element_type=jnp.float32,
        )
        + b_ref[...].T
    )


def _tc_matmul(x, W, b2d):
    grid = (pl.cdiv(VOCAB, BV),)
    return pl.pallas_call(
        _mm_kernel,
        grid=grid,
        in_specs=[
            pl.BlockSpec((BB, EMB), lambda j: (0, 0)),
            pl.BlockSpec((BV, EMB), lambda j: (j, 0)),
            pl.BlockSpec((1, BV), lambda j: (0, j)),
        ],
        out_specs=pl.BlockSpec((BV, BB), lambda j: (j, 0)),
        out_shape=jax.ShapeDtypeStruct((VOCAB, B), jnp.float32),
    )(x, W, b2d)


def kernel(inputs, emb_table, W, b):
    idx_grouped = inputs.astype(jnp.int32).reshape(NW * NG, G * L)
    summed = _sc_gather_sum(idx_grouped, emb_table)
    out_t = _tc_matmul(summed, W, b.reshape(1, VOCAB))
    return out_t.T
